# baseline (speedup 1.0000x reference)
import jax
import jax.numpy as jnp
from jax.experimental import pallas as pl

BATCH_BLOCK = 256


def _bcast_kernel(pe_ref, out_ref):
    out_ref[...] = jnp.broadcast_to(pe_ref[...][None, :, :], out_ref.shape)


def kernel(x, pe_weight):
    batch = x.shape[0]
    max_len, dim = pe_weight.shape
    grid = (batch // BATCH_BLOCK,)
    return pl.pallas_call(
        _bcast_kernel,
        grid=grid,
        in_specs=[pl.BlockSpec((max_len, dim), lambda i: (0, 0))],
        out_specs=pl.BlockSpec((BATCH_BLOCK, max_len, dim), lambda i: (i, 0, 0)),
        out_shape=jax.ShapeDtypeStruct((batch, max_len, dim), pe_weight.dtype),
    )(pe_weight)


# speedup vs baseline: 1.6583x; 1.6583x over previous
import jax
import jax.numpy as jnp
from jax.experimental import pallas as pl

BATCH_BLOCK = 128


def _bcast_kernel(pe_ref, out_ref):
    out_ref[...] = jnp.broadcast_to(pe_ref[...], out_ref.shape)


def kernel(x, pe_weight):
    batch = x.shape[0]
    max_len, dim = pe_weight.shape
    flat = max_len * dim
    pe_flat = pe_weight.reshape(1, flat)
    out2d = pl.pallas_call(
        _bcast_kernel,
        grid=(batch // BATCH_BLOCK,),
        in_specs=[pl.BlockSpec((1, flat), lambda i: (0, 0))],
        out_specs=pl.BlockSpec((BATCH_BLOCK, flat), lambda i: (i, 0)),
        out_shape=jax.ShapeDtypeStruct((batch, flat), pe_weight.dtype),
    )(pe_flat)
    return out2d.reshape(batch, max_len, dim)


# speedup vs baseline: 1.6708x; 1.0075x over previous
import jax
import jax.numpy as jnp
from jax.experimental import pallas as pl
from jax.experimental.pallas import tpu as pltpu

REP = 128


def _make_copy_kernel(n_copies):
    def _copy_kernel(pe_ref, out_ref, buf_ref, sem):
        buf_ref[...] = jnp.broadcast_to(pe_ref[...], buf_ref.shape)
        copies = [
            pltpu.make_async_copy(
                buf_ref, out_ref.at[pl.ds(i * REP, REP)], sem
            )
            for i in range(n_copies)
        ]
        for c in copies:
            c.start()
        for c in copies:
            c.wait()

    return _copy_kernel


def kernel(x, pe_weight):
    batch = x.shape[0]
    max_len, dim = pe_weight.shape
    flat = max_len * dim
    pe_flat = pe_weight.reshape(1, flat)
    n_copies = batch // REP
    out2d = pl.pallas_call(
        _make_copy_kernel(n_copies),
        in_specs=[pl.BlockSpec(memory_space=pltpu.MemorySpace.VMEM)],
        out_specs=pl.BlockSpec(memory_space=pl.ANY),
        out_shape=jax.ShapeDtypeStruct((batch, flat), pe_weight.dtype),
        scratch_shapes=[
            pltpu.VMEM((REP, flat), pe_weight.dtype),
            pltpu.SemaphoreType.DMA,
        ],
    )(pe_flat)
    return out2d.reshape(batch, max_len, dim)
